# baseline (speedup 1.0000x reference)
import functools

import jax
import jax.numpy as jnp
from jax import lax
from jax.experimental import pallas as pl
from jax.experimental.pallas import tpu as pltpu
from jax.experimental.pallas import tpu_sc as plsc

N = 10000
E = 160000
FEAT = 128
H = 32
EHID = 32
STEPS = 6
S2S_STEPS = 6
S2S_LAYERS = 3
OUT = 32

NP = 10240
EP = 163840
NW = 32
EPW = EP // NW
CHUNK = 128
NCH = EPW // CHUNK
ROWS_BUF = 128
NPS = NP // 16
WIDE = 128


def _lin0_body(nf_ref, w_ref, b_ref, o_ref):
    r = jax.nn.relu(
        jnp.dot(nf_ref[...], w_ref[...], preferred_element_type=jnp.float32)
        + b_ref[...])
    o_ref[...] = jnp.concatenate(
        [r, jnp.zeros((r.shape[0], WIDE - H), jnp.float32)], axis=1)


def _lin0(nf, wT, b):
    blk = 2048
    return pl.pallas_call(
        _lin0_body,
        grid=(NP // blk,),
        in_specs=[
            pl.BlockSpec((blk, nf.shape[1]), lambda i: (i, 0)),
            pl.BlockSpec(wT.shape, lambda i: (0, 0)),
            pl.BlockSpec(b.shape, lambda i: (0, 0)),
        ],
        out_specs=pl.BlockSpec((blk, WIDE), lambda i: (i, 0)),
        out_shape=jax.ShapeDtypeStruct((NP, WIDE), jnp.float32),
    )(nf, wT, b)


def _edge_mlp_body(ea_ref, w_ref, b_ref, o_ref):
    o_ref[...] = jax.nn.relu(
        jnp.dot(ea_ref[...], w_ref[...], preferred_element_type=jnp.float32)
        + b_ref[...])


def _edge_mlp(ea, wT, b):
    blk = 2048
    return pl.pallas_call(
        _edge_mlp_body,
        grid=(EP // blk,),
        in_specs=[
            pl.BlockSpec((blk, ea.shape[1]), lambda i: (i, 0)),
            pl.BlockSpec(wT.shape, lambda i: (0, 0)),
            pl.BlockSpec(b.shape, lambda i: (0, 0)),
        ],
        out_specs=pl.BlockSpec((blk, H), lambda i: (i, 0)),
        out_shape=jax.ShapeDtypeStruct((EP, H), jnp.float32),
    )(ea, wT, b)


def _msg_body(h2_ref, u_ref, m_ref, o_ref):
    h2 = h2_ref[...]
    u = u_ref[:, :H]
    pieces = [h2[:, k:k + 1] * u for k in range(EHID)]
    pieces.append(u)
    o = jnp.concatenate(pieces, axis=1)
    r = jnp.dot(o, m_ref[...], preferred_element_type=jnp.float32)
    o_ref[...] = jnp.concatenate(
        [r, jnp.zeros((r.shape[0], WIDE - H), jnp.float32)], axis=1)


def _msg(h2, u, m33):
    blk = 1024
    return pl.pallas_call(
        _msg_body,
        grid=(EP // blk,),
        in_specs=[
            pl.BlockSpec((blk, H), lambda i: (i, 0)),
            pl.BlockSpec((blk, WIDE), lambda i: (i, 0)),
            pl.BlockSpec(m33.shape, lambda i: (0, 0)),
        ],
        out_specs=pl.BlockSpec((blk, WIDE), lambda i: (i, 0)),
        out_shape=jax.ShapeDtypeStruct((EP, WIDE), jnp.float32),
    )(h2, u, m33)


def _gru_body(a0_ref, a1_ref, h_ref, cb_ref, wih_ref, whh_ref, bih_ref,
              bhh_ref, o_ref):
    m = jax.nn.relu(a0_ref[:, :H] + a1_ref[:, :H] + cb_ref[...])
    h = h_ref[:, :H]
    gi = jnp.dot(m, wih_ref[...], preferred_element_type=jnp.float32) + bih_ref[...]
    gh = jnp.dot(h, whh_ref[...], preferred_element_type=jnp.float32) + bhh_ref[...]
    r = jax.nn.sigmoid(gi[:, :H] + gh[:, :H])
    z = jax.nn.sigmoid(gi[:, H:2 * H] + gh[:, H:2 * H])
    n = jnp.tanh(gi[:, 2 * H:] + r * gh[:, 2 * H:])
    o = (1.0 - z) * n + z * h
    o_ref[...] = jnp.concatenate(
        [o, jnp.zeros((o.shape[0], WIDE - H), jnp.float32)], axis=1)


def _gru(a0, a1, h, cb, wihT, whhT, bih, bhh):
    blk = 2048
    return pl.pallas_call(
        _gru_body,
        grid=(NP // blk,),
        in_specs=[
            pl.BlockSpec((blk, WIDE), lambda i: (i, 0)),
            pl.BlockSpec((blk, WIDE), lambda i: (i, 0)),
            pl.BlockSpec((blk, WIDE), lambda i: (i, 0)),
            pl.BlockSpec(cb.shape, lambda i: (0, 0)),
            pl.BlockSpec(wihT.shape, lambda i: (0, 0)),
            pl.BlockSpec(whhT.shape, lambda i: (0, 0)),
            pl.BlockSpec(bih.shape, lambda i: (0, 0)),
            pl.BlockSpec(bhh.shape, lambda i: (0, 0)),
        ],
        out_specs=pl.BlockSpec((blk, WIDE), lambda i: (i, 0)),
        out_shape=jax.ShapeDtypeStruct((NP, WIDE), jnp.float32),
    )(a0, a1, h, cb, wihT, whhT, bih, bhh)


def _s2s_body(out_ref, w0i_ref, w0h_ref, b0_ref, w1i_ref, w1h_ref, b1_ref,
              w2i_ref, w2h_ref, b2_ref, r1_ref, r1b_ref, r2_ref, r2b_ref,
              y_ref):
    out = out_ref[:, :H]
    row = lax.broadcasted_iota(jnp.int32, (NP, 1), 0)
    valid = row < N
    wi = (w0i_ref, w1i_ref, w2i_ref)
    wh = (w0h_ref, w1h_ref, w2h_ref)
    bb = (b0_ref, b1_ref, b2_ref)
    hs = [jnp.zeros((1, H), jnp.float32) for _ in range(S2S_LAYERS)]
    cs = [jnp.zeros((1, H), jnp.float32) for _ in range(S2S_LAYERS)]
    q_star = jnp.zeros((1, 2 * H), jnp.float32)
    for _ in range(S2S_STEPS):
        x = q_star
        for l in range(S2S_LAYERS):
            gates = (jnp.dot(x, wi[l][...], preferred_element_type=jnp.float32)
                     + jnp.dot(hs[l], wh[l][...],
                               preferred_element_type=jnp.float32)
                     + bb[l][...])
            i_g = jax.nn.sigmoid(gates[:, :H])
            f_g = jax.nn.sigmoid(gates[:, H:2 * H])
            g_g = jnp.tanh(gates[:, 2 * H:3 * H])
            o_g = jax.nn.sigmoid(gates[:, 3 * H:])
            cs[l] = f_g * cs[l] + i_g * g_g
            hs[l] = o_g * jnp.tanh(cs[l])
            x = hs[l]
        q = x
        e = jnp.sum(out * q, axis=1, keepdims=True)
        e = jnp.where(valid, e, -jnp.inf)
        e_max = jnp.max(e)
        a = jnp.where(valid, jnp.exp(e - e_max), 0.0)
        denom = jnp.sum(a)
        readout = jnp.sum(a * out, axis=0, keepdims=True) / denom
        q_star = jnp.concatenate([q, readout], axis=1)
    t = jax.nn.relu(
        jnp.dot(q_star, r1_ref[...], preferred_element_type=jnp.float32)
        + r1b_ref[...])
    y_ref[...] = (jnp.dot(t, r2_ref[...], preferred_element_type=jnp.float32)
                  + r2b_ref[...])


def _s2s(out, lstm_wT, r1T, r1b, r2T, r2b):
    args = [out]
    for l in range(S2S_LAYERS):
        args.extend(lstm_wT[l])
    args.extend([r1T, r1b, r2T, r2b])
    return pl.pallas_call(
        _s2s_body,
        out_shape=jax.ShapeDtypeStruct((1, OUT), jnp.float32),
    )(*args)


def _sc_gather(table, idx2):
    mesh = plsc.VectorSubcoreMesh(core_axis_name="core",
                                  subcore_axis_name="subcore")

    @functools.partial(
        pl.kernel,
        out_type=jax.ShapeDtypeStruct((EP, WIDE), jnp.float32),
        mesh=mesh)
    def k(x_hbm, i_hbm, o_hbm):
        def body(i_vmem, o_vmem):
            pltpu.sync_copy(x_hbm.at[i_vmem.at[0]], o_vmem)

        pltpu.emit_pipeline(
            body,
            grid=(EP // CHUNK,),
            in_specs=[pl.BlockSpec((1, CHUNK), index_map=lambda i: (0, i))],
            out_specs=[pl.BlockSpec((CHUNK, WIDE), index_map=lambda i: (i, 0))],
            core_axis_name=("core", "subcore"),
            dimension_semantics=(pltpu.PARALLEL,),
        )(i_hbm, o_hbm)

    return k(table, idx2)


def _sc_scatter_add(msg, dst3, zrows):
    mesh = plsc.VectorSubcoreMesh(core_axis_name="core",
                                  subcore_axis_name="subcore")

    @functools.partial(
        pl.kernel,
        out_type=jax.ShapeDtypeStruct((2, NP, WIDE), jnp.float32),
        mesh=mesh,
        scratch_types=[
            pltpu.VMEM_SHARED((NP, WIDE), jnp.float32),
            pltpu.VMEM((NCH, CHUNK), jnp.int32),
            pltpu.VMEM((ROWS_BUF, WIDE), jnp.float32),
        ])
    def k(msg_hbm, dst_hbm, z_hbm, o_hbm, shared, idx_v, rows_v):
        cid = lax.axis_index("core")
        sid = lax.axis_index("subcore")
        w = sid * 2 + cid
        pltpu.sync_copy(z_hbm.at[pl.ds(sid * NPS, NPS)],
                        shared.at[pl.ds(sid * NPS, NPS)])
        plsc.subcore_barrier()
        pltpu.sync_copy(dst_hbm.at[w], idx_v)
        n_super = EPW // ROWS_BUF
        per = ROWS_BUF // CHUNK

        @pl.loop(0, n_super)
        def _(sc):
            pltpu.sync_copy(msg_hbm.at[pl.ds(w * EPW + sc * ROWS_BUF,
                                             ROWS_BUF)], rows_v)
            for j in range(per):
                pltpu.sync_copy(rows_v.at[pl.ds(j * CHUNK, CHUNK)],
                                shared.at[idx_v.at[sc * per + j]],
                                add=True)

        plsc.subcore_barrier()
        pltpu.sync_copy(shared.at[pl.ds(sid * NPS, NPS)],
                        o_hbm.at[cid, pl.ds(sid * NPS, NPS)])

    return k(msg, dst3, zrows)


def kernel(features, edge_attr, seed, edge_index, lin0_W, lin0_b, eW1, eb1,
           eW2, eb2, conv_b, gru_Wih, gru_Whh, gru_bih, gru_bhh,
           lstm_Wih0, lstm_Whh0, lstm_bih0, lstm_bhh0,
           lstm_Wih1, lstm_Whh1, lstm_bih1, lstm_bhh1,
           lstm_Wih2, lstm_Whh2, lstm_bih2, lstm_bhh2,
           r1W, r1b, r2W, r2b):
    f32 = jnp.float32
    nf = jnp.concatenate([features, seed[:, None].astype(f32)], axis=1)
    nf = jnp.pad(nf, ((0, NP - N), (0, 0)))
    ea = jnp.pad(edge_attr, ((0, EP - E), (0, 0)))
    src = jnp.pad(edge_index[0].astype(jnp.int32), (0, EP - E))
    dst = jnp.pad(edge_index[1].astype(jnp.int32), (0, EP - E),
                  constant_values=N)
    idx2 = src.reshape(1, EP)
    dst3 = dst.reshape(NW, NCH, CHUNK)
    zrows = jnp.zeros((NP, WIDE), f32)

    lin0T = lin0_W.T
    lin0b = lin0_b.reshape(1, H)
    eW1T = eW1.T
    eb1r = eb1.reshape(1, EHID)
    a3 = eW2.reshape(H, H, EHID)
    mk = a3.transpose(2, 0, 1).reshape(EHID * H, H)
    m33 = jnp.concatenate([mk, eb2.reshape(H, H)], axis=0)
    cbr = conv_b.reshape(1, H)
    gwihT = gru_Wih.T
    gwhhT = gru_Whh.T
    gbih = gru_bih.reshape(1, 3 * H)
    gbhh = gru_bhh.reshape(1, 3 * H)
    lstm_wT = [
        (lstm_Wih0.T, lstm_Whh0.T, (lstm_bih0 + lstm_bhh0).reshape(1, 4 * H)),
        (lstm_Wih1.T, lstm_Whh1.T, (lstm_bih1 + lstm_bhh1).reshape(1, 4 * H)),
        (lstm_Wih2.T, lstm_Whh2.T, (lstm_bih2 + lstm_bhh2).reshape(1, 4 * H)),
    ]
    r1T = r1W.T
    r1br = r1b.reshape(1, H)
    r2T = r2W.T
    r2br = r2b.reshape(1, OUT)

    out = _lin0(nf, lin0T, lin0b)
    h2 = _edge_mlp(ea, eW1T, eb1r)
    h = out
    for _ in range(STEPS):
        u = _sc_gather(h, idx2)
        msg = _msg(h2, u, m33)
        agg = _sc_scatter_add(msg, dst3, zrows)
        h = _gru(agg[0], agg[1], h, cbr, gwihT, gwhhT, gbih, gbhh)
    y = _s2s(h, lstm_wT, r1T, r1br, r2T, r2br)
    return y


# speedup vs baseline: 2.7118x; 2.7118x over previous
import functools

import jax
import jax.numpy as jnp
from jax import lax
from jax.experimental import pallas as pl
from jax.experimental.pallas import tpu as pltpu
from jax.experimental.pallas import tpu_sc as plsc

N = 10000
E = 160000
FEAT = 128
H = 32
EHID = 32
STEPS = 6
S2S_STEPS = 6
S2S_LAYERS = 3
OUT = 32

NP = 10240
EP = 163840
NW = 32
EPW = EP // NW
CHUNK = 128
NCH = EPW // CHUNK
ROWS_BUF = 128
NPS = NP // 16
WIDE = 128


def _lin0_body(nf_ref, w_ref, b_ref, o_ref):
    r = jax.nn.relu(
        jnp.dot(nf_ref[...], w_ref[...], preferred_element_type=jnp.float32)
        + b_ref[...])
    o_ref[...] = jnp.concatenate(
        [r, jnp.zeros((r.shape[0], WIDE - H), jnp.float32)], axis=1)


def _lin0(nf, wT, b):
    blk = 2048
    return pl.pallas_call(
        _lin0_body,
        grid=(NP // blk,),
        in_specs=[
            pl.BlockSpec((blk, nf.shape[1]), lambda i: (i, 0)),
            pl.BlockSpec(wT.shape, lambda i: (0, 0)),
            pl.BlockSpec(b.shape, lambda i: (0, 0)),
        ],
        out_specs=pl.BlockSpec((blk, WIDE), lambda i: (i, 0)),
        out_shape=jax.ShapeDtypeStruct((NP, WIDE), jnp.float32),
    )(nf, wT, b)


def _edge_mlp_body(eaT_ref, w_ref, b_ref, o_ref):
    o_ref[...] = jax.nn.relu(
        jnp.dot(w_ref[...], eaT_ref[...], preferred_element_type=jnp.float32)
        + b_ref[...])


def _edge_mlp_t(eaT, w, b):
    blk = 2048
    return pl.pallas_call(
        _edge_mlp_body,
        grid=(EP // blk,),
        in_specs=[
            pl.BlockSpec((eaT.shape[0], blk), lambda i: (0, i)),
            pl.BlockSpec(w.shape, lambda i: (0, 0)),
            pl.BlockSpec(b.shape, lambda i: (0, 0)),
        ],
        out_specs=pl.BlockSpec((EHID, blk), lambda i: (0, i)),
        out_shape=jax.ShapeDtypeStruct((EHID, EP), jnp.float32),
    )(eaT, w, b)


def _msg_body(h2T_ref, u_ref, mT_ref, o_ref):
    blk = u_ref.shape[0]
    h2T = h2T_ref[...]
    uT = jnp.transpose(u_ref[:, :H])
    prod = h2T[:, None, :] * uT[None, :, :]
    ot = jnp.concatenate(
        [prod.reshape(EHID * H, blk), uT], axis=0)
    msgT = jnp.dot(mT_ref[...], ot, preferred_element_type=jnp.float32)
    r = jnp.transpose(msgT)
    o_ref[...] = jnp.concatenate(
        [r, jnp.zeros((blk, WIDE - H), jnp.float32)], axis=1)


def _msg(h2T, u, m33T):
    blk = 1024
    return pl.pallas_call(
        _msg_body,
        grid=(EP // blk,),
        in_specs=[
            pl.BlockSpec((EHID, blk), lambda i: (0, i)),
            pl.BlockSpec((blk, WIDE), lambda i: (i, 0)),
            pl.BlockSpec(m33T.shape, lambda i: (0, 0)),
        ],
        out_specs=pl.BlockSpec((blk, WIDE), lambda i: (i, 0)),
        out_shape=jax.ShapeDtypeStruct((EP, WIDE), jnp.float32),
    )(h2T, u, m33T)


def _gru_body(a0_ref, a1_ref, h_ref, cb_ref, wih_ref, whh_ref, bih_ref,
              bhh_ref, o_ref):
    m = jax.nn.relu(a0_ref[:, :H] + a1_ref[:, :H] + cb_ref[...])
    h = h_ref[:, :H]
    gi = jnp.dot(m, wih_ref[...], preferred_element_type=jnp.float32) + bih_ref[...]
    gh = jnp.dot(h, whh_ref[...], preferred_element_type=jnp.float32) + bhh_ref[...]
    r = jax.nn.sigmoid(gi[:, :H] + gh[:, :H])
    z = jax.nn.sigmoid(gi[:, H:2 * H] + gh[:, H:2 * H])
    n = jnp.tanh(gi[:, 2 * H:] + r * gh[:, 2 * H:])
    o = (1.0 - z) * n + z * h
    o_ref[...] = jnp.concatenate(
        [o, jnp.zeros((o.shape[0], WIDE - H), jnp.float32)], axis=1)


def _gru(a0, a1, h, cb, wihT, whhT, bih, bhh):
    blk = 2048
    return pl.pallas_call(
        _gru_body,
        grid=(NP // blk,),
        in_specs=[
            pl.BlockSpec((blk, WIDE), lambda i: (i, 0)),
            pl.BlockSpec((blk, WIDE), lambda i: (i, 0)),
            pl.BlockSpec((blk, WIDE), lambda i: (i, 0)),
            pl.BlockSpec(cb.shape, lambda i: (0, 0)),
            pl.BlockSpec(wihT.shape, lambda i: (0, 0)),
            pl.BlockSpec(whhT.shape, lambda i: (0, 0)),
            pl.BlockSpec(bih.shape, lambda i: (0, 0)),
            pl.BlockSpec(bhh.shape, lambda i: (0, 0)),
        ],
        out_specs=pl.BlockSpec((blk, WIDE), lambda i: (i, 0)),
        out_shape=jax.ShapeDtypeStruct((NP, WIDE), jnp.float32),
    )(a0, a1, h, cb, wihT, whhT, bih, bhh)


def _s2s_body(out_ref, w0i_ref, w0h_ref, b0_ref, w1i_ref, w1h_ref, b1_ref,
              w2i_ref, w2h_ref, b2_ref, r1_ref, r1b_ref, r2_ref, r2b_ref,
              y_ref):
    out = out_ref[:, :H]
    row = lax.broadcasted_iota(jnp.int32, (NP, 1), 0)
    valid = row < N
    wi = (w0i_ref, w1i_ref, w2i_ref)
    wh = (w0h_ref, w1h_ref, w2h_ref)
    bb = (b0_ref, b1_ref, b2_ref)
    hs = [jnp.zeros((1, H), jnp.float32) for _ in range(S2S_LAYERS)]
    cs = [jnp.zeros((1, H), jnp.float32) for _ in range(S2S_LAYERS)]
    q_star = jnp.zeros((1, 2 * H), jnp.float32)
    for _ in range(S2S_STEPS):
        x = q_star
        for l in range(S2S_LAYERS):
            gates = (jnp.dot(x, wi[l][...], preferred_element_type=jnp.float32)
                     + jnp.dot(hs[l], wh[l][...],
                               preferred_element_type=jnp.float32)
                     + bb[l][...])
            i_g = jax.nn.sigmoid(gates[:, :H])
            f_g = jax.nn.sigmoid(gates[:, H:2 * H])
            g_g = jnp.tanh(gates[:, 2 * H:3 * H])
            o_g = jax.nn.sigmoid(gates[:, 3 * H:])
            cs[l] = f_g * cs[l] + i_g * g_g
            hs[l] = o_g * jnp.tanh(cs[l])
            x = hs[l]
        q = x
        e = jnp.sum(out * q, axis=1, keepdims=True)
        e = jnp.where(valid, e, -jnp.inf)
        e_max = jnp.max(e)
        a = jnp.where(valid, jnp.exp(e - e_max), 0.0)
        denom = jnp.sum(a)
        readout = jnp.sum(a * out, axis=0, keepdims=True) / denom
        q_star = jnp.concatenate([q, readout], axis=1)
    t = jax.nn.relu(
        jnp.dot(q_star, r1_ref[...], preferred_element_type=jnp.float32)
        + r1b_ref[...])
    y_ref[...] = (jnp.dot(t, r2_ref[...], preferred_element_type=jnp.float32)
                  + r2b_ref[...])


def _s2s(out, lstm_wT, r1T, r1b, r2T, r2b):
    args = [out]
    for l in range(S2S_LAYERS):
        args.extend(lstm_wT[l])
    args.extend([r1T, r1b, r2T, r2b])
    return pl.pallas_call(
        _s2s_body,
        out_shape=jax.ShapeDtypeStruct((1, OUT), jnp.float32),
    )(*args)


def _sc_gather(table, idx3):
    mesh = plsc.VectorSubcoreMesh(core_axis_name="core",
                                  subcore_axis_name="subcore")

    @functools.partial(
        pl.kernel,
        out_type=jax.ShapeDtypeStruct((EP, WIDE), jnp.float32),
        mesh=mesh,
        scratch_types=[
            pltpu.VMEM((NCH, CHUNK), jnp.int32),
            pltpu.VMEM((CHUNK, WIDE), jnp.float32),
            pltpu.VMEM((CHUNK, WIDE), jnp.float32),
            pltpu.SemaphoreType.DMA,
            pltpu.SemaphoreType.DMA,
        ])
    def k(x_hbm, i_hbm, o_hbm, idx_v, buf0, buf1, sg, ss):
        cid = lax.axis_index("core")
        sid = lax.axis_index("subcore")
        w = sid * 2 + cid
        pltpu.sync_copy(i_hbm.at[w], idx_v)

        @pl.loop(0, NCH // 2)
        def _(jj):
            j = jj * 2
            base = w * EPW + j * CHUNK
            g0 = pltpu.async_copy(x_hbm.at[idx_v.at[j]], buf0, sg)
            g1 = pltpu.async_copy(x_hbm.at[idx_v.at[j + 1]], buf1, sg)
            g0.wait()
            s0 = pltpu.async_copy(buf0, o_hbm.at[pl.ds(base, CHUNK)], ss)
            g1.wait()
            s1 = pltpu.async_copy(
                buf1, o_hbm.at[pl.ds(base + CHUNK, CHUNK)], ss)
            s0.wait()
            s1.wait()

    return k(table, idx3)


def _sc_scatter_add(msg, dst3, zrows):
    mesh = plsc.VectorSubcoreMesh(core_axis_name="core",
                                  subcore_axis_name="subcore")

    @functools.partial(
        pl.kernel,
        out_type=jax.ShapeDtypeStruct((2, NP, WIDE), jnp.float32),
        mesh=mesh,
        scratch_types=[
            pltpu.VMEM_SHARED((NP, WIDE), jnp.float32),
            pltpu.VMEM((NCH, CHUNK), jnp.int32),
            pltpu.VMEM((CHUNK, WIDE), jnp.float32),
            pltpu.VMEM((CHUNK, WIDE), jnp.float32),
            pltpu.SemaphoreType.DMA,
        ])
    def k(msg_hbm, dst_hbm, z_hbm, o_hbm, shared, idx_v, rows0, rows1, sl):
        cid = lax.axis_index("core")
        sid = lax.axis_index("subcore")
        w = sid * 2 + cid
        pltpu.sync_copy(z_hbm.at[pl.ds(sid * NPS, NPS)],
                        shared.at[pl.ds(sid * NPS, NPS)])
        plsc.subcore_barrier()
        pltpu.sync_copy(dst_hbm.at[w], idx_v)

        @pl.loop(0, NCH // 2)
        def _(jj):
            j = jj * 2
            base = w * EPW + j * CHUNK
            l0 = pltpu.async_copy(msg_hbm.at[pl.ds(base, CHUNK)], rows0, sl)
            l1 = pltpu.async_copy(
                msg_hbm.at[pl.ds(base + CHUNK, CHUNK)], rows1, sl)
            l0.wait()
            pltpu.sync_copy(rows0, shared.at[idx_v.at[j]], add=True)
            l1.wait()
            pltpu.sync_copy(rows1, shared.at[idx_v.at[j + 1]], add=True)

        plsc.subcore_barrier()
        pltpu.sync_copy(shared.at[pl.ds(sid * NPS, NPS)],
                        o_hbm.at[cid, pl.ds(sid * NPS, NPS)])

    return k(msg, dst3, zrows)


def kernel(features, edge_attr, seed, edge_index, lin0_W, lin0_b, eW1, eb1,
           eW2, eb2, conv_b, gru_Wih, gru_Whh, gru_bih, gru_bhh,
           lstm_Wih0, lstm_Whh0, lstm_bih0, lstm_bhh0,
           lstm_Wih1, lstm_Whh1, lstm_bih1, lstm_bhh1,
           lstm_Wih2, lstm_Whh2, lstm_bih2, lstm_bhh2,
           r1W, r1b, r2W, r2b):
    f32 = jnp.float32
    nf = jnp.concatenate([features, seed[:, None].astype(f32)], axis=1)
    nf = jnp.pad(nf, ((0, NP - N), (0, 0)))
    eaT = jnp.pad(edge_attr, ((0, EP - E), (0, 0))).T
    src = jnp.pad(edge_index[0].astype(jnp.int32), (0, EP - E))
    dst = jnp.pad(edge_index[1].astype(jnp.int32), (0, EP - E),
                  constant_values=N)
    idx3 = src.reshape(NW, NCH, CHUNK)
    dst3 = dst.reshape(NW, NCH, CHUNK)
    zrows = jnp.zeros((NP, WIDE), f32)

    lin0T = lin0_W.T
    lin0b = lin0_b.reshape(1, H)
    eb1r = eb1.reshape(EHID, 1)
    a3 = eW2.reshape(H, H, EHID)
    mk = a3.transpose(2, 0, 1).reshape(EHID * H, H)
    m33T = jnp.concatenate([mk, eb2.reshape(H, H)], axis=0).T
    cbr = conv_b.reshape(1, H)
    gwihT = gru_Wih.T
    gwhhT = gru_Whh.T
    gbih = gru_bih.reshape(1, 3 * H)
    gbhh = gru_bhh.reshape(1, 3 * H)
    lstm_wT = [
        (lstm_Wih0.T, lstm_Whh0.T, (lstm_bih0 + lstm_bhh0).reshape(1, 4 * H)),
        (lstm_Wih1.T, lstm_Whh1.T, (lstm_bih1 + lstm_bhh1).reshape(1, 4 * H)),
        (lstm_Wih2.T, lstm_Whh2.T, (lstm_bih2 + lstm_bhh2).reshape(1, 4 * H)),
    ]
    r1T = r1W.T
    r1br = r1b.reshape(1, H)
    r2T = r2W.T
    r2br = r2b.reshape(1, OUT)

    out = _lin0(nf, lin0T, lin0b)
    h2T = _edge_mlp_t(eaT, eW1, eb1r)
    h = out
    for _ in range(STEPS):
        u = _sc_gather(h, idx3)
        msg = _msg(h2T, u, m33T)
        agg = _sc_scatter_add(msg, dst3, zrows)
        h = _gru(agg[0], agg[1], h, cbr, gwihT, gwhhT, gbih, gbhh)
    y = _s2s(h, lstm_wT, r1T, r1br, r2T, r2br)
    return y


# speedup vs baseline: 4.6221x; 1.7044x over previous
import functools

import jax
import jax.numpy as jnp
from jax import lax
from jax.experimental import pallas as pl
from jax.experimental.pallas import tpu as pltpu
from jax.experimental.pallas import tpu_sc as plsc

N = 10000
E = 160000
FEAT = 128
H = 32
EHID = 32
STEPS = 6
S2S_STEPS = 6
S2S_LAYERS = 3
OUT = 32

NP = 10240
EP = 163840
NW = 32
EPW = EP // NW
CHUNK = 128
NCH = EPW // CHUNK
ROWS_BUF = 128
NPS = NP // 16
WIDE = 128


def _lin0_body(nf_ref, w_ref, b_ref, o_ref):
    r = jax.nn.relu(
        jnp.dot(nf_ref[...], w_ref[...], preferred_element_type=jnp.float32)
        + b_ref[...])
    o_ref[...] = jnp.concatenate(
        [r, jnp.zeros((r.shape[0], WIDE - H), jnp.float32)], axis=1)


def _lin0(nf, wT, b):
    blk = 2048
    return pl.pallas_call(
        _lin0_body,
        grid=(NP // blk,),
        in_specs=[
            pl.BlockSpec((blk, nf.shape[1]), lambda i: (i, 0)),
            pl.BlockSpec(wT.shape, lambda i: (0, 0)),
            pl.BlockSpec(b.shape, lambda i: (0, 0)),
        ],
        out_specs=pl.BlockSpec((blk, WIDE), lambda i: (i, 0)),
        out_shape=jax.ShapeDtypeStruct((NP, WIDE), jnp.float32),
    )(nf, wT, b)


def _edge_mlp_body(eaT_ref, w_ref, b_ref, o_ref):
    o_ref[...] = jax.nn.relu(
        jnp.dot(w_ref[...], eaT_ref[...], preferred_element_type=jnp.float32)
        + b_ref[...])


def _edge_mlp_t(eaT, w, b):
    blk = 2048
    return pl.pallas_call(
        _edge_mlp_body,
        grid=(EP // blk,),
        in_specs=[
            pl.BlockSpec((eaT.shape[0], blk), lambda i: (0, i)),
            pl.BlockSpec(w.shape, lambda i: (0, 0)),
            pl.BlockSpec(b.shape, lambda i: (0, 0)),
        ],
        out_specs=pl.BlockSpec((EHID, blk), lambda i: (0, i)),
        out_shape=jax.ShapeDtypeStruct((EHID, EP), jnp.float32),
    )(eaT, w, b)


def _msg_body(h2T_ref, u_ref, mT_ref, o_ref):
    blk = u_ref.shape[0]
    h2T = h2T_ref[...]
    uT = jnp.transpose(u_ref[:, :H])
    prod = h2T[:, None, :] * uT[None, :, :]
    ot = jnp.concatenate(
        [prod.reshape(EHID * H, blk), uT], axis=0)
    msgT = jnp.dot(mT_ref[...], ot, preferred_element_type=jnp.float32)
    r = jnp.transpose(msgT)
    o_ref[...] = jnp.concatenate(
        [r, jnp.zeros((blk, WIDE - H), jnp.float32)], axis=1)


def _msg(h2T, u, m33T):
    blk = 1024
    return pl.pallas_call(
        _msg_body,
        grid=(EP // blk,),
        in_specs=[
            pl.BlockSpec((EHID, blk), lambda i: (0, i)),
            pl.BlockSpec((blk, WIDE), lambda i: (i, 0)),
            pl.BlockSpec(m33T.shape, lambda i: (0, 0)),
        ],
        out_specs=pl.BlockSpec((blk, WIDE), lambda i: (i, 0)),
        out_shape=jax.ShapeDtypeStruct((EP, WIDE), jnp.float32),
    )(h2T, u, m33T)


def _gru_body(a0_ref, a1_ref, h_ref, cb_ref, wih_ref, whh_ref, bih_ref,
              bhh_ref, o_ref):
    m = jax.nn.relu(a0_ref[:, :H] + a1_ref[:, :H] + cb_ref[...])
    h = h_ref[:, :H]
    gi = jnp.dot(m, wih_ref[...], preferred_element_type=jnp.float32) + bih_ref[...]
    gh = jnp.dot(h, whh_ref[...], preferred_element_type=jnp.float32) + bhh_ref[...]
    r = jax.nn.sigmoid(gi[:, :H] + gh[:, :H])
    z = jax.nn.sigmoid(gi[:, H:2 * H] + gh[:, H:2 * H])
    n = jnp.tanh(gi[:, 2 * H:] + r * gh[:, 2 * H:])
    o = (1.0 - z) * n + z * h
    o_ref[...] = jnp.concatenate(
        [o, jnp.zeros((o.shape[0], WIDE - H), jnp.float32)], axis=1)


def _gru(a0, a1, h, cb, wihT, whhT, bih, bhh):
    blk = 2048
    return pl.pallas_call(
        _gru_body,
        grid=(NP // blk,),
        in_specs=[
            pl.BlockSpec((blk, WIDE), lambda i: (i, 0)),
            pl.BlockSpec((blk, WIDE), lambda i: (i, 0)),
            pl.BlockSpec((blk, WIDE), lambda i: (i, 0)),
            pl.BlockSpec(cb.shape, lambda i: (0, 0)),
            pl.BlockSpec(wihT.shape, lambda i: (0, 0)),
            pl.BlockSpec(whhT.shape, lambda i: (0, 0)),
            pl.BlockSpec(bih.shape, lambda i: (0, 0)),
            pl.BlockSpec(bhh.shape, lambda i: (0, 0)),
        ],
        out_specs=pl.BlockSpec((blk, WIDE), lambda i: (i, 0)),
        out_shape=jax.ShapeDtypeStruct((NP, WIDE), jnp.float32),
    )(a0, a1, h, cb, wihT, whhT, bih, bhh)


def _s2s_body(out_ref, w0i_ref, w0h_ref, b0_ref, w1i_ref, w1h_ref, b1_ref,
              w2i_ref, w2h_ref, b2_ref, r1_ref, r1b_ref, r2_ref, r2b_ref,
              y_ref):
    out = out_ref[:, :H]
    row = lax.broadcasted_iota(jnp.int32, (NP, 1), 0)
    valid = row < N
    wi = (w0i_ref, w1i_ref, w2i_ref)
    wh = (w0h_ref, w1h_ref, w2h_ref)
    bb = (b0_ref, b1_ref, b2_ref)
    hs = [jnp.zeros((1, H), jnp.float32) for _ in range(S2S_LAYERS)]
    cs = [jnp.zeros((1, H), jnp.float32) for _ in range(S2S_LAYERS)]
    q_star = jnp.zeros((1, 2 * H), jnp.float32)
    for _ in range(S2S_STEPS):
        x = q_star
        for l in range(S2S_LAYERS):
            gates = (jnp.dot(x, wi[l][...], preferred_element_type=jnp.float32)
                     + jnp.dot(hs[l], wh[l][...],
                               preferred_element_type=jnp.float32)
                     + bb[l][...])
            i_g = jax.nn.sigmoid(gates[:, :H])
            f_g = jax.nn.sigmoid(gates[:, H:2 * H])
            g_g = jnp.tanh(gates[:, 2 * H:3 * H])
            o_g = jax.nn.sigmoid(gates[:, 3 * H:])
            cs[l] = f_g * cs[l] + i_g * g_g
            hs[l] = o_g * jnp.tanh(cs[l])
            x = hs[l]
        q = x
        e = jnp.sum(out * q, axis=1, keepdims=True)
        e = jnp.where(valid, e, -jnp.inf)
        e_max = jnp.max(e)
        a = jnp.where(valid, jnp.exp(e - e_max), 0.0)
        denom = jnp.sum(a)
        readout = jnp.sum(a * out, axis=0, keepdims=True) / denom
        q_star = jnp.concatenate([q, readout], axis=1)
    t = jax.nn.relu(
        jnp.dot(q_star, r1_ref[...], preferred_element_type=jnp.float32)
        + r1b_ref[...])
    y_ref[...] = (jnp.dot(t, r2_ref[...], preferred_element_type=jnp.float32)
                  + r2b_ref[...])


def _s2s(out, lstm_wT, r1T, r1b, r2T, r2b):
    args = [out]
    for l in range(S2S_LAYERS):
        args.extend(lstm_wT[l])
    args.extend([r1T, r1b, r2T, r2b])
    return pl.pallas_call(
        _s2s_body,
        out_shape=jax.ShapeDtypeStruct((1, OUT), jnp.float32),
    )(*args)


def _sc_gather(table, idx3):
    mesh = plsc.VectorSubcoreMesh(core_axis_name="core",
                                  subcore_axis_name="subcore")

    @functools.partial(
        pl.kernel,
        out_type=jax.ShapeDtypeStruct((EP, WIDE), jnp.float32),
        mesh=mesh,
        scratch_types=[
            pltpu.VMEM_SHARED((NP, WIDE), jnp.float32),
            pltpu.VMEM((NCH, CHUNK), jnp.int32),
            pltpu.VMEM((CHUNK, WIDE), jnp.float32),
            pltpu.VMEM((CHUNK, WIDE), jnp.float32),
            pltpu.SemaphoreType.DMA,
            pltpu.SemaphoreType.DMA,
        ])
    def k(x_hbm, i_hbm, o_hbm, shared, idx_v, buf0, buf1, sg, ss):
        cid = lax.axis_index("core")
        sid = lax.axis_index("subcore")
        w = sid * 2 + cid
        pltpu.sync_copy(x_hbm.at[pl.ds(sid * NPS, NPS)],
                        shared.at[pl.ds(sid * NPS, NPS)])
        pltpu.sync_copy(i_hbm.at[w], idx_v)
        plsc.subcore_barrier()

        @pl.loop(0, NCH // 2)
        def _(jj):
            j = jj * 2
            base = w * EPW + j * CHUNK
            g0 = pltpu.async_copy(shared.at[idx_v.at[j]], buf0, sg)
            g1 = pltpu.async_copy(shared.at[idx_v.at[j + 1]], buf1, sg)
            g0.wait()
            s0 = pltpu.async_copy(buf0, o_hbm.at[pl.ds(base, CHUNK)], ss)
            g1.wait()
            s1 = pltpu.async_copy(
                buf1, o_hbm.at[pl.ds(base + CHUNK, CHUNK)], ss)
            s0.wait()
            s1.wait()

    return k(table, idx3)


def _sc_scatter_add(msg, dst3, zrows):
    mesh = plsc.VectorSubcoreMesh(core_axis_name="core",
                                  subcore_axis_name="subcore")

    @functools.partial(
        pl.kernel,
        out_type=jax.ShapeDtypeStruct((2, NP, WIDE), jnp.float32),
        mesh=mesh,
        scratch_types=[
            pltpu.VMEM_SHARED((NP, WIDE), jnp.float32),
            pltpu.VMEM((NCH, CHUNK), jnp.int32),
            pltpu.VMEM((CHUNK, WIDE), jnp.float32),
            pltpu.VMEM((CHUNK, WIDE), jnp.float32),
            pltpu.SemaphoreType.DMA,
        ])
    def k(msg_hbm, dst_hbm, z_hbm, o_hbm, shared, idx_v, rows0, rows1, sl):
        cid = lax.axis_index("core")
        sid = lax.axis_index("subcore")
        w = sid * 2 + cid
        pltpu.sync_copy(z_hbm.at[pl.ds(sid * NPS, NPS)],
                        shared.at[pl.ds(sid * NPS, NPS)])
        plsc.subcore_barrier()
        pltpu.sync_copy(dst_hbm.at[w], idx_v)

        @pl.loop(0, NCH // 2)
        def _(jj):
            j = jj * 2
            base = w * EPW + j * CHUNK
            l0 = pltpu.async_copy(msg_hbm.at[pl.ds(base, CHUNK)], rows0, sl)
            l1 = pltpu.async_copy(
                msg_hbm.at[pl.ds(base + CHUNK, CHUNK)], rows1, sl)
            l0.wait()
            pltpu.sync_copy(rows0, shared.at[idx_v.at[j]], add=True)
            l1.wait()
            pltpu.sync_copy(rows1, shared.at[idx_v.at[j + 1]], add=True)

        plsc.subcore_barrier()
        pltpu.sync_copy(shared.at[pl.ds(sid * NPS, NPS)],
                        o_hbm.at[cid, pl.ds(sid * NPS, NPS)])

    return k(msg, dst3, zrows)


def kernel(features, edge_attr, seed, edge_index, lin0_W, lin0_b, eW1, eb1,
           eW2, eb2, conv_b, gru_Wih, gru_Whh, gru_bih, gru_bhh,
           lstm_Wih0, lstm_Whh0, lstm_bih0, lstm_bhh0,
           lstm_Wih1, lstm_Whh1, lstm_bih1, lstm_bhh1,
           lstm_Wih2, lstm_Whh2, lstm_bih2, lstm_bhh2,
           r1W, r1b, r2W, r2b):
    f32 = jnp.float32
    nf = jnp.concatenate([features, seed[:, None].astype(f32)], axis=1)
    nf = jnp.pad(nf, ((0, NP - N), (0, 0)))
    eaT = jnp.pad(edge_attr, ((0, EP - E), (0, 0))).T
    src = jnp.pad(edge_index[0].astype(jnp.int32), (0, EP - E))
    dst = jnp.pad(edge_index[1].astype(jnp.int32), (0, EP - E),
                  constant_values=N)
    idx3 = src.reshape(NW, NCH, CHUNK)
    dst3 = dst.reshape(NW, NCH, CHUNK)
    zrows = jnp.zeros((NP, WIDE), f32)

    lin0T = lin0_W.T
    lin0b = lin0_b.reshape(1, H)
    eb1r = eb1.reshape(EHID, 1)
    a3 = eW2.reshape(H, H, EHID)
    mk = a3.transpose(2, 0, 1).reshape(EHID * H, H)
    m33T = jnp.concatenate([mk, eb2.reshape(H, H)], axis=0).T
    cbr = conv_b.reshape(1, H)
    gwihT = gru_Wih.T
    gwhhT = gru_Whh.T
    gbih = gru_bih.reshape(1, 3 * H)
    gbhh = gru_bhh.reshape(1, 3 * H)
    lstm_wT = [
        (lstm_Wih0.T, lstm_Whh0.T, (lstm_bih0 + lstm_bhh0).reshape(1, 4 * H)),
        (lstm_Wih1.T, lstm_Whh1.T, (lstm_bih1 + lstm_bhh1).reshape(1, 4 * H)),
        (lstm_Wih2.T, lstm_Whh2.T, (lstm_bih2 + lstm_bhh2).reshape(1, 4 * H)),
    ]
    r1T = r1W.T
    r1br = r1b.reshape(1, H)
    r2T = r2W.T
    r2br = r2b.reshape(1, OUT)

    out = _lin0(nf, lin0T, lin0b)
    h2T = _edge_mlp_t(eaT, eW1, eb1r)
    h = out
    for _ in range(STEPS):
        u = _sc_gather(h, idx3)
        msg = _msg(h2T, u, m33T)
        agg = _sc_scatter_add(msg, dst3, zrows)
        h = _gru(agg[0], agg[1], h, cbr, gwihT, gwhhT, gbih, gbhh)
    y = _s2s(h, lstm_wT, r1T, r1br, r2T, r2br)
    return y
